# baseline (speedup 1.0000x reference)
import functools

import jax
import jax.numpy as jnp
import numpy as np
from jax.experimental import pallas as pl
from jax.experimental.pallas import tpu as pltpu
from jax.experimental.pallas import tpu_sc as plsc

_HIDDEN = 1024
_S = 2
_BITS = 8
_R = _HIDDEN // _BITS
_M = 4
_J = _S * _R
_VOCAB = 1 << (2 * _BITS)


def _pack_matrix():
    f = np.arange(_S * _HIDDEN)
    s, rem = f // _HIDDEN, f % _HIDDEN
    r, k = rem // _BITS, rem % _BITS
    p = np.zeros((_S * _HIDDEN, _J), np.float32)
    p[f, s * _R + r] = np.exp2(k)
    return jnp.asarray(p, jnp.bfloat16)


def _idx_body(h_ref, w_ref, p_ref, o_ref):
    h = h_ref[0]
    q = jnp.dot(h.astype(jnp.bfloat16), w_ref[...],
                preferred_element_type=jnp.float32)
    bits = (q > 0).astype(jnp.bfloat16)
    codes = jnp.dot(bits, p_ref[...],
                    preferred_element_type=jnp.float32).astype(jnp.int32)
    t = codes.shape[0]
    prev = jnp.concatenate(
        [jnp.zeros((1, _J), jnp.int32), codes[:-1, :]], axis=0)
    j16 = jax.lax.broadcasted_iota(jnp.int32, (t, _J), 1) << 16
    o_ref[0] = j16 + prev + (codes << 8)


def _out_body(h_ref, m_ref, w_ref, o_ref, *, blk, t):
    i = pl.program_id(0)
    rows = i * blk + jax.lax.broadcasted_iota(jnp.int32, (blk, _HIDDEN), 0)
    m = jnp.where((rows % t) == 0, 0.0, m_ref[...])
    inj = jnp.dot(m.astype(jnp.bfloat16), w_ref[...],
                  preferred_element_type=jnp.float32)
    o_ref[...] = h_ref[...] + inj


def _gather_call(bank, gidx_flat, n, window):
    bank16 = jax.lax.bitcast_convert_type(
        bank, jnp.bfloat16).reshape(bank.shape[0], 2 * _M)
    mesh = plsc.VectorSubcoreMesh(
        core_axis_name="core", subcore_axis_name="subcore")

    @functools.partial(
        pl.kernel,
        out_type=jax.ShapeDtypeStruct((n, 2 * _M), jnp.bfloat16),
        mesh=mesh,
        compiler_params=pltpu.CompilerParams(use_tc_tiling_on_sc=False))
    def _gather(bank_hbm, idx_hbm, out_hbm):
        def body(i_vmem, o_vmem):
            pltpu.sync_copy(bank_hbm.at[i_vmem.at[0]], o_vmem)

        pltpu.emit_pipeline(
            body,
            grid=(n // window,),
            in_specs=[pl.BlockSpec((1, window), lambda i: (0, i))],
            out_specs=[pl.BlockSpec((window, 2 * _M), lambda i: (i, 0))],
            core_axis_name=("core", "subcore"),
            dimension_semantics=(pltpu.PARALLEL,),
        )(idx_hbm, out_hbm)

    out16 = _gather(bank16, gidx_flat)
    return jax.lax.bitcast_convert_type(
        out16.reshape(n, _M, 2), jnp.float32)


def kernel(hidden_states, latent_q_weight, bank, out_proj):
    b, t, h = hidden_states.shape
    fused_t = latent_q_weight.reshape(_S * h, h).T.astype(jnp.bfloat16)
    pmat = _pack_matrix()
    out_proj_t = out_proj.T.astype(jnp.bfloat16)

    gidx = pl.pallas_call(
        _idx_body,
        grid=(b,),
        in_specs=[
            pl.BlockSpec((1, t, h), lambda i: (i, 0, 0)),
            pl.BlockSpec((h, _S * h), lambda i: (0, 0)),
            pl.BlockSpec((_S * h, _J), lambda i: (0, 0)),
        ],
        out_specs=pl.BlockSpec((1, t, _J), lambda i: (i, 0, 0)),
        out_shape=jax.ShapeDtypeStruct((b, t, _J), jnp.int32),
    )(hidden_states, fused_t, pmat)

    n = b * t * _J
    mem = _gather_call(bank, gidx.reshape(1, n), n, 128)
    mem_flat = mem.reshape(b * t, _J * _M)

    blk = 512
    hidden_flat = hidden_states.reshape(b * t, h)
    out = pl.pallas_call(
        functools.partial(_out_body, blk=blk, t=t),
        grid=(b * t // blk,),
        in_specs=[
            pl.BlockSpec((blk, h), lambda i: (i, 0)),
            pl.BlockSpec((blk, h), lambda i: (i, 0)),
            pl.BlockSpec((h, h), lambda i: (0, 0)),
        ],
        out_specs=pl.BlockSpec((blk, h), lambda i: (i, 0)),
        out_shape=jax.ShapeDtypeStruct((b * t, h), jnp.float32),
    )(hidden_flat, mem_flat, out_proj_t)
    return out.reshape(b, t, h)
